# baseline (speedup 1.0000x reference)
import functools

import jax
import jax.numpy as jnp
from jax import lax
from jax.experimental import pallas as pl
from jax.experimental.pallas import tpu as pltpu
from jax.experimental.pallas import tpu_sc as plsc

NFIELD = 26
NEMB = 16
K = 16
HID = 64
B = 4096

NC, NS = 2, 16
NW = NC * NS
FPAD = 32
NG = FPAD // 8
ROWS = B * FPAD
CH = 128
NCHUNK = ROWS // (NW * CH)
DIN = FPAD * NEMB


def _sc_gather(xi, si, xtab, stab):
    mesh = plsc.VectorSubcoreMesh(
        core_axis_name="c", subcore_axis_name="s",
        num_cores=NC, num_subcores=NS)

    @functools.partial(
        pl.kernel,
        out_type=(
            jax.ShapeDtypeStruct((NW, NCHUNK, CH, NEMB), jnp.float32),
            jax.ShapeDtypeStruct((NW, NCHUNK, CH, NEMB), jnp.float32),
        ),
        mesh=mesh,
        scratch_types=[
            pltpu.VMEM((NCHUNK, CH), jnp.int32),
            pltpu.VMEM((NCHUNK, CH, NEMB), jnp.float32),
            pltpu.SemaphoreType.DMA,
        ],
        compiler_params=pltpu.CompilerParams(use_tc_tiling_on_sc=False),
    )
    def gather_kernel(xi_hbm, si_hbm, xtab_hbm, stab_hbm, xout_hbm, sout_hbm,
                      idx_v, rows_v, sem):
        wid = lax.axis_index("s") * NC + lax.axis_index("c")

        pltpu.sync_copy(xi_hbm.at[wid], idx_v)

        @pl.loop(0, NCHUNK)
        def _fire_x(j):
            pltpu.async_copy(xtab_hbm.at[idx_v.at[j]], rows_v.at[j], sem)

        pltpu.make_async_copy(xout_hbm.at[wid], rows_v, sem).wait()
        pltpu.sync_copy(rows_v, xout_hbm.at[wid])

        pltpu.sync_copy(si_hbm.at[wid], idx_v)

        @pl.loop(0, NCHUNK)
        def _fire_s(j):
            pltpu.async_copy(stab_hbm.at[idx_v.at[j]], rows_v.at[j], sem)

        pltpu.make_async_copy(sout_hbm.at[wid], rows_v, sem).wait()
        pltpu.sync_copy(rows_v, sout_hbm.at[wid])

    return gather_kernel(xi, si, xtab, stab)


def _tc_fused(xg, sg, gw1p, gb1, gw2, gb2, w1p, b1c, w2blk, eb2r):
    T = 8
    BT = B // T
    KH = K * HID

    def body(x0_ref, x1_ref, x2_ref, x3_ref, s0_ref, s1_ref, s2_ref, s3_ref,
             gw1_ref, gb1_ref, gw2_ref, gb2_ref,
             w1_ref, b1_ref, w2_ref, eb2_ref, y_ref, loss_ref, acc_ref):
        t = pl.program_id(0)

        se = jnp.concatenate(
            [s0_ref[...], s1_ref[...], s2_ref[...], s3_ref[...]], axis=-1)
        gh = jnp.maximum(
            jnp.dot(se, gw1_ref[...], preferred_element_type=jnp.float32)
            + gb1_ref[...], 0.0)
        gl = (jnp.dot(gh, gw2_ref[...], preferred_element_type=jnp.float32)
              + gb2_ref[...])
        gm = jnp.max(gl, axis=-1, keepdims=True)
        ge = jnp.exp(gl - gm)
        p = ge / jnp.sum(ge, axis=-1, keepdims=True)

        col = lax.broadcasted_iota(jnp.int32, p.shape, 1)
        v1 = jnp.max(p, axis=-1, keepdims=True)
        i1 = jnp.min(jnp.where(p == v1, col, K), axis=-1, keepdims=True)
        mask1 = col == i1
        pm = jnp.where(mask1, -1.0, p)
        v2 = jnp.max(pm, axis=-1, keepdims=True)
        i2 = jnp.min(jnp.where(pm == v2, col, K), axis=-1, keepdims=True)
        mask2 = col == i2
        gates = (jnp.where(mask1, v1, 0.0)
                 + jnp.where(mask2, v2, 0.0))

        xe = jnp.concatenate(
            [x0_ref[...], x1_ref[...], x2_ref[...], x3_ref[...]], axis=-1)
        h = jnp.maximum(
            jnp.dot(xe, w1_ref[...], preferred_element_type=jnp.float32)
            + b1_ref[...], 0.0)
        eo = (jnp.dot(h, w2_ref[...], preferred_element_type=jnp.float32)
              + eb2_ref[...])
        y_ref[...] = jnp.sum(gates * eo, axis=-1, keepdims=True)

        @pl.when(t == 0)
        def _init():
            acc_ref[...] = jnp.zeros_like(acc_ref)

        acc_ref[...] += jnp.sum(gates, axis=0, keepdims=True)

        @pl.when(t == T - 1)
        def _fin():
            imp = acc_ref[...]
            mean = jnp.sum(imp, axis=-1, keepdims=True) / K
            var = jnp.sum((imp - mean) ** 2, axis=-1, keepdims=True) / K
            loss_ref[...] = var / (mean * mean + 1e-10)

    panel = pl.BlockSpec((BT, 128), lambda t: (t, 0))
    y, loss = pl.pallas_call(
        body,
        grid=(T,),
        in_specs=[
            panel, panel, panel, panel,
            panel, panel, panel, panel,
            pl.BlockSpec((DIN, HID), lambda t: (0, 0)),
            pl.BlockSpec((1, HID), lambda t: (0, 0)),
            pl.BlockSpec((HID, K), lambda t: (0, 0)),
            pl.BlockSpec((1, K), lambda t: (0, 0)),
            pl.BlockSpec((DIN, KH), lambda t: (0, 0)),
            pl.BlockSpec((1, KH), lambda t: (0, 0)),
            pl.BlockSpec((KH, K), lambda t: (0, 0)),
            pl.BlockSpec((1, K), lambda t: (0, 0)),
        ],
        out_specs=[
            pl.BlockSpec((BT, 1), lambda t: (t, 0)),
            pl.BlockSpec((1, 1), lambda t: (0, 0)),
        ],
        out_shape=[
            jax.ShapeDtypeStruct((B, 1), jnp.float32),
            jax.ShapeDtypeStruct((1, 1), jnp.float32),
        ],
        scratch_shapes=[pltpu.VMEM((1, K), jnp.float32)],
    )(*xg, *sg, gw1p, gb1, gw2, gb2, w1p, b1c, w2blk, eb2r)
    return y, loss


def _group_major_idx(a):
    ap = jnp.pad(a, ((0, 0), (0, FPAD - NFIELD)))
    return ap.reshape(B, NG, 8).transpose(1, 0, 2).reshape(NW, NCHUNK, CH)


def kernel(x, sql, sql_table, input_table, gw1, gb1, gw2, gb2, ew1, eb1, ew2, eb2):
    xi = _group_major_idx(x)
    si = _group_major_idx(sql)
    xr, sr = _sc_gather(xi, si, input_table, sql_table)
    xfull = xr.reshape(NG, B, 128)
    sfull = sr.reshape(NG, B, 128)
    xg = [xfull[q] for q in range(NG)]
    sg = [sfull[q] for q in range(NG)]

    zpad = jnp.zeros((DIN - NFIELD * NEMB, HID), dtype=jnp.float32)
    gw1p = jnp.concatenate([gw1, zpad], axis=0)
    w1c = ew1.transpose(1, 0, 2).reshape(NFIELD * NEMB, K * HID)
    zpad2 = jnp.zeros((DIN - NFIELD * NEMB, K * HID), dtype=jnp.float32)
    w1p = jnp.concatenate([w1c, zpad2], axis=0)
    b1c = eb1.reshape(1, K * HID)
    eyek = jnp.eye(K, dtype=jnp.float32)
    w2blk = (ew2[:, :, 0][:, :, None] * eyek[:, None, :]).reshape(K * HID, K)
    eb2r = eb2.reshape(1, K)

    y, loss = _tc_fused(xg, sg, gw1p, gb1.reshape(1, HID),
                        gw2, gb2.reshape(1, K), w1p, b1c, w2blk, eb2r)
    return (y.reshape(B), loss.reshape(()))
